# Initial kernel scaffold; baseline (speedup 1.0000x reference)
import jax
import jax.numpy as jnp
from jax.experimental import pallas as pl


def kernel(Xa, Xb, Fin, ND):
    raise NotImplementedError("write your pallas kernel here")


# speedup vs baseline: 14.6705x; 14.6705x over previous
import functools

import jax
import jax.numpy as jnp
from jax import lax
from jax.experimental import pallas as pl
from jax.experimental.pallas import tpu as pltpu
from jax.experimental.pallas import tpu_sc as plsc

K = 8
A = 8192
R = 2048
NN = 16
F = 256
NC, NS, L = 2, 16, 16
NW = NC * NS
WPB = NW // K
RPW = R // WPB
G = 8
NG = RPW // G
FC = F // L


def _body(xa, xb, nd, fin, out, xa_v, xb_v, nd_v, rows_v, out_v, gg_v, sem):
    wid = lax.axis_index("s") * NC + lax.axis_index("c")
    k = wid // WPB
    r0 = (wid % WPB) * RPW
    pltpu.sync_copy(xa.at[k, pl.ds(0, A * 3)], xa_v)
    pltpu.sync_copy(xb.at[k, pl.ds(r0 * 3, RPW * 3)], xb_v)
    pltpu.sync_copy(nd.at[k, pl.ds(r0 * NN, RPW * NN)], nd_v)
    k8 = k * A

    def absbody(i, _):
        nd_v[pl.ds(i * L, L)] = nd_v[pl.ds(i * L, L)] + k8
        return 0

    lax.fori_loop(0, RPW * NN // L, absbody, 0)

    cz = jnp.zeros((L,), jnp.int32)
    c1 = jnp.full((L,), 1, jnp.int32)
    c2 = jnp.full((L,), 2, jnp.int32)

    def group(g, _):
        base = g * G
        pltpu.async_copy(fin.at[nd_v.at[pl.ds(base * NN, G * NN)]], rows_v, sem).wait()
        for rr in range(G):
            r = base + rr
            idx3 = (nd_v[pl.ds(r * NN, NN)] - k8) * 3
            x0 = plsc.load_gather(xa_v, [idx3])
            x1 = plsc.load_gather(xa_v, [idx3 + c1])
            x2 = plsc.load_gather(xa_v, [idx3 + c2])
            rsplat = jnp.full((L,), r * 3, jnp.int32)
            d0 = x0 - plsc.load_gather(xb_v, [rsplat])
            d1 = x1 - plsc.load_gather(xb_v, [rsplat + c1])
            d2 = x2 - plsc.load_gather(xb_v, [rsplat + c2])
            dsq = d0 * d0 + d1 * d1 + d2 * d2
            om = jnp.max(dsq, axis=0)
            gg = jnp.exp(dsq / om)
            s = jnp.sum(gg, axis=0)
            gg_v[...] = gg / s

            def nbody(n, accs):
                w = plsc.load_gather(gg_v, [jnp.full((L,), n, jnp.int32)])
                row = rr * NN + n
                return tuple(
                    accs[c] + w * rows_v[row, pl.ds(c * L, L)] for c in range(FC)
                )

            accs = lax.fori_loop(
                0, NN, nbody, tuple(jnp.zeros((L,), jnp.float32) for _ in range(FC))
            )
            for c in range(FC):
                out_v[rr, pl.ds(c * L, L)] = accs[c]
        pltpu.sync_copy(out_v, out.at[k, pl.ds(r0 + base, G)])
        return 0

    lax.fori_loop(0, NG, group, 0)


@functools.partial(
    pl.kernel,
    out_type=jax.ShapeDtypeStruct((K, R, F), jnp.float32),
    mesh=plsc.VectorSubcoreMesh(core_axis_name="c", subcore_axis_name="s"),
    compiler_params=pltpu.CompilerParams(needs_layout_passes=False),
    scratch_types=[
        pltpu.VMEM((A * 3,), jnp.float32),
        pltpu.VMEM((RPW * 3,), jnp.float32),
        pltpu.VMEM((RPW * NN,), jnp.int32),
        pltpu.VMEM((G * NN, F), jnp.float32),
        pltpu.VMEM((G, F), jnp.float32),
        pltpu.VMEM((NN,), jnp.float32),
        pltpu.SemaphoreType.DMA,
    ],
)
def _sc_kernel(xa, xb, nd, fin, out, *rest):
    _body(xa, xb, nd, fin, out, *rest)


def kernel(Xa, Xb, Fin, ND):
    xa2 = Xa.reshape(K, A * 3)
    xb2 = Xb.reshape(K, R * 3)
    nd2 = ND.reshape(K, R * NN)
    fin2 = Fin.reshape(K * A, F)
    return _sc_kernel(xa2, xb2, nd2, fin2)


# speedup vs baseline: 17.5044x; 1.1932x over previous
import functools

import jax
import jax.numpy as jnp
from jax import lax
from jax.experimental import pallas as pl
from jax.experimental.pallas import tpu as pltpu
from jax.experimental.pallas import tpu_sc as plsc

K = 8
A = 8192
R = 2048
NN = 16
F = 256
NC, NS, L = 2, 16, 16
NW = NC * NS
WPB = NW // K
RPW = R // WPB
G = 8
NG = RPW // G
FC = F // L
UN = 4


def _body(xa, xb, nd, fin, out, xa_v, xb_v, nd_v, rows0, rows1, out_v, gg_v,
          sem0, sem1):
    wid = lax.axis_index("s") * NC + lax.axis_index("c")
    k = wid // WPB
    r0 = (wid % WPB) * RPW
    pltpu.sync_copy(xa.at[k, pl.ds(0, A * 3)], xa_v)
    pltpu.sync_copy(xb.at[k, pl.ds(r0 * 3, RPW * 3)], xb_v)
    pltpu.sync_copy(nd.at[k, pl.ds(r0 * NN, RPW * NN)], nd_v)
    k8 = k * A

    def absbody(i, _):
        nd_v[pl.ds(i * L, L)] = nd_v[pl.ds(i * L, L)] + k8
        return 0

    lax.fori_loop(0, RPW * NN // L, absbody, 0)

    c1 = jnp.full((L,), 1, jnp.int32)
    c2 = jnp.full((L,), 2, jnp.int32)

    def start(g, rows_v, sem):
        pltpu.async_copy(fin.at[nd_v.at[pl.ds(g * G * NN, G * NN)]], rows_v, sem)

    def wait(rows_v, sem):
        pltpu.make_async_copy(fin.at[pl.ds(0, G)], rows_v, sem).wait()

    def compute_group(g, rows_v):
        base = g * G
        for rr in range(G):
            r = base + rr
            idx3 = (nd_v[pl.ds(r * NN, NN)] - k8) * 3
            x0 = plsc.load_gather(xa_v, [idx3])
            x1 = plsc.load_gather(xa_v, [idx3 + c1])
            x2 = plsc.load_gather(xa_v, [idx3 + c2])
            rsplat = jnp.full((L,), r * 3, jnp.int32)
            d0 = x0 - plsc.load_gather(xb_v, [rsplat])
            d1 = x1 - plsc.load_gather(xb_v, [rsplat + c1])
            d2 = x2 - plsc.load_gather(xb_v, [rsplat + c2])
            dsq = d0 * d0 + d1 * d1 + d2 * d2
            om = jnp.max(dsq, axis=0)
            gg = jnp.exp(dsq / om)
            s = jnp.sum(gg, axis=0)
            gg_v[...] = gg / s

            def nbody(j, accs):
                accs = list(accs)
                for dn in range(UN):
                    n = j * UN + dn
                    w = plsc.load_gather(gg_v, [jnp.full((L,), n, jnp.int32)])
                    row = rr * NN + n
                    accs = [
                        accs[c] + w * rows_v[row, pl.ds(c * L, L)]
                        for c in range(FC)
                    ]
                return tuple(accs)

            accs = lax.fori_loop(
                0, NN // UN, nbody,
                tuple(jnp.zeros((L,), jnp.float32) for _ in range(FC)),
            )
            for c in range(FC):
                out_v[rr, pl.ds(c * L, L)] = accs[c]
        pltpu.sync_copy(out_v, out.at[k, pl.ds(r0 + base, G)])

    start(0, rows0, sem0)

    def pair(h, _):
        g0 = 2 * h
        wait(rows0, sem0)
        start(g0 + 1, rows1, sem1)
        compute_group(g0, rows0)
        wait(rows1, sem1)

        @pl.when(h < NG // 2 - 1)
        def _():
            start(g0 + 2, rows0, sem0)

        compute_group(g0 + 1, rows1)
        return 0

    lax.fori_loop(0, NG // 2, pair, 0)


@functools.partial(
    pl.kernel,
    out_type=jax.ShapeDtypeStruct((K, R, F), jnp.float32),
    mesh=plsc.VectorSubcoreMesh(core_axis_name="c", subcore_axis_name="s"),
    compiler_params=pltpu.CompilerParams(needs_layout_passes=False),
    scratch_types=[
        pltpu.VMEM((A * 3,), jnp.float32),
        pltpu.VMEM((RPW * 3,), jnp.float32),
        pltpu.VMEM((RPW * NN,), jnp.int32),
        pltpu.VMEM((G * NN, F), jnp.float32),
        pltpu.VMEM((G * NN, F), jnp.float32),
        pltpu.VMEM((G, F), jnp.float32),
        pltpu.VMEM((NN,), jnp.float32),
        pltpu.SemaphoreType.DMA,
        pltpu.SemaphoreType.DMA,
    ],
)
def _sc_kernel(xa, xb, nd, fin, out, *rest):
    _body(xa, xb, nd, fin, out, *rest)


def kernel(Xa, Xb, Fin, ND):
    xa2 = Xa.reshape(K, A * 3)
    xb2 = Xb.reshape(K, R * 3)
    nd2 = ND.reshape(K, R * NN)
    fin2 = Fin.reshape(K * A, F)
    return _sc_kernel(xa2, xb2, nd2, fin2)


# speedup vs baseline: 18.4294x; 1.0528x over previous
import functools

import jax
import jax.numpy as jnp
from jax import lax
from jax.experimental import pallas as pl
from jax.experimental.pallas import tpu as pltpu
from jax.experimental.pallas import tpu_sc as plsc

K = 8
A = 8192
R = 2048
NN = 16
F = 256
NC, NS, L = 2, 16, 16
NW = NC * NS
WPB = NW // K
RPW = R // WPB
G = 8
NG = RPW // G
FC = F // L
UN = 4
NB = 2


def _body(xa, xb, nd, fin, out, xa_v, xb_v, nd_v, rows_v, out_v, gg_v,
          sem_in, sem_out):
    wid = lax.axis_index("s") * NC + lax.axis_index("c")
    k = wid // WPB
    r0 = (wid % WPB) * RPW
    pltpu.sync_copy(xa.at[k, pl.ds(0, A * 3)], xa_v)
    pltpu.sync_copy(xb.at[k, pl.ds(r0 * 3, RPW * 3)], xb_v)
    pltpu.sync_copy(nd.at[k, pl.ds(r0 * NN, RPW * NN)], nd_v)
    k8 = k * A

    def absbody(i, _):
        for u in range(4):
            j = i * 4 + u
            nd_v[pl.ds(j * L, L)] = nd_v[pl.ds(j * L, L)] + k8
        return 0

    lax.fori_loop(0, RPW * NN // (4 * L), absbody, 0)

    c1 = jnp.full((L,), 1, jnp.int32)
    c2 = jnp.full((L,), 2, jnp.int32)

    def start(g, p):
        pltpu.async_copy(
            fin.at[nd_v.at[pl.ds(g * G * NN, G * NN)]],
            rows_v.at[pl.ds(p * G * NN, G * NN)],
            sem_in,
        )

    start(0, 0)
    start(1, 1)

    def group(g, _):
        p = lax.rem(g, NB)
        pltpu.make_async_copy(
            fin.at[pl.ds(0, G * NN)], rows_v.at[pl.ds(0, G * NN)], sem_in
        ).wait()

        @pl.when(g >= NB)
        def _():
            pltpu.make_async_copy(
                out_v.at[pl.ds(0, G)], out.at[k, pl.ds(r0, G)], sem_out
            ).wait()

        base = g * G
        poff = p * G * NN
        for rr in range(G):
            r = base + rr
            idx3 = (nd_v[pl.ds(r * NN, NN)] - k8) * 3
            x0 = plsc.load_gather(xa_v, [idx3])
            x1 = plsc.load_gather(xa_v, [idx3 + c1])
            x2 = plsc.load_gather(xa_v, [idx3 + c2])
            rsplat = jnp.full((L,), r * 3, jnp.int32)
            d0 = x0 - plsc.load_gather(xb_v, [rsplat])
            d1 = x1 - plsc.load_gather(xb_v, [rsplat + c1])
            d2 = x2 - plsc.load_gather(xb_v, [rsplat + c2])
            dsq = d0 * d0 + d1 * d1 + d2 * d2
            om = jnp.max(dsq, axis=0)
            gg = jnp.exp(dsq / om)
            s = jnp.sum(gg, axis=0)
            gg_v[...] = gg / s

            def nbody(j, accs):
                accs = list(accs)
                for dn in range(UN):
                    n = j * UN + dn
                    w = plsc.load_gather(gg_v, [jnp.full((L,), n, jnp.int32)])
                    row = poff + rr * NN + n
                    accs = [
                        accs[c] + w * rows_v[row, pl.ds(c * L, L)]
                        for c in range(FC)
                    ]
                return tuple(accs)

            accs = lax.fori_loop(
                0, NN // UN, nbody,
                tuple(jnp.zeros((L,), jnp.float32) for _ in range(FC)),
            )
            for c in range(FC):
                out_v[p * G + rr, pl.ds(c * L, L)] = accs[c]
        pltpu.async_copy(
            out_v.at[pl.ds(p * G, G)], out.at[k, pl.ds(r0 + base, G)], sem_out
        )

        @pl.when(g + NB < NG)
        def _():
            start(g + NB, p)

        return 0

    lax.fori_loop(0, NG, group, 0)
    for _ in range(NB):
        pltpu.make_async_copy(
            out_v.at[pl.ds(0, G)], out.at[k, pl.ds(r0, G)], sem_out
        ).wait()


@functools.partial(
    pl.kernel,
    out_type=jax.ShapeDtypeStruct((K, R, F), jnp.float32),
    mesh=plsc.VectorSubcoreMesh(core_axis_name="c", subcore_axis_name="s"),
    compiler_params=pltpu.CompilerParams(needs_layout_passes=False),
    scratch_types=[
        pltpu.VMEM((A * 3,), jnp.float32),
        pltpu.VMEM((RPW * 3,), jnp.float32),
        pltpu.VMEM((RPW * NN,), jnp.int32),
        pltpu.VMEM((NB * G * NN, F), jnp.float32),
        pltpu.VMEM((NB * G, F), jnp.float32),
        pltpu.VMEM((NN,), jnp.float32),
        pltpu.SemaphoreType.DMA,
        pltpu.SemaphoreType.DMA,
    ],
)
def _sc_kernel(xa, xb, nd, fin, out, *rest):
    _body(xa, xb, nd, fin, out, *rest)


def kernel(Xa, Xb, Fin, ND):
    xa2 = Xa.reshape(K, A * 3)
    xb2 = Xb.reshape(K, R * 3)
    nd2 = ND.reshape(K, R * NN)
    fin2 = Fin.reshape(K * A, F)
    return _sc_kernel(xa2, xb2, nd2, fin2)
